# baseline (speedup 1.0000x reference)
import jax
import jax.numpy as jnp
from jax.experimental import pallas as pl
from jax.experimental.pallas import tpu as pltpu

TS = 16
B, N, H, W = 2, 8, 384, 384
C = 96
K = 128
TW = W // TS
GW = 3
BH = 32
NBLK = H // BH
TR = BH // TS


def _dense_body(act8_ref, ls_ref, mask_ref, ff_ref, out_ref):
    b = pl.program_id(0)
    hb = pl.program_id(1)
    x = ls_ref[0]
    strength = jnp.maximum(x, 0.0) + jnp.log(1.0 + jnp.exp(-jnp.abs(x)))
    sig = jnp.tanh(jnp.sum(ff_ref[0], axis=0) * (1.0 / C))
    j = jax.lax.broadcasted_iota(jnp.int32, (TS, 128), 1) // TS
    for tr in range(TR):
        for g in range(GW):
            bits = act8_ref[b, (hb * TR + tr) * GW + g]
            rs = slice(tr * TS, (tr + 1) * TS)
            cs = slice(g * 128, (g + 1) * 128)
            active = ((bits >> j) & 1) == 1
            delta = jnp.where(active, strength * sig[rs, cs], 0.0)
            out_ref[0, :, rs, cs] = mask_ref[0, :, rs, cs] + delta[None, :, :]


def kernel(mask_logits, ff_highres_features, log_strength, active_tile_indices):
    idx = jnp.asarray(active_tile_indices, jnp.int32)
    ls = jnp.asarray(log_strength, jnp.float32).reshape(1)
    bb = jnp.arange(B, dtype=jnp.int32)[:, None]
    tile_act = jnp.zeros((B, (H // TS) * TW), jnp.int32).at[bb, idx].max(1)
    act8 = jnp.sum(
        tile_act.reshape(B, H // TS, GW, 8) << jnp.arange(8, dtype=jnp.int32),
        axis=-1,
    ).reshape(B, (H // TS) * GW)

    return pl.pallas_call(
        _dense_body,
        grid=(B, NBLK),
        in_specs=[
            pl.BlockSpec(memory_space=pltpu.SMEM),
            pl.BlockSpec(memory_space=pltpu.SMEM),
            pl.BlockSpec((1, N, BH, W), lambda b, hb: (b, 0, hb, 0)),
            pl.BlockSpec((1, C, BH, W), lambda b, hb: (b, 0, hb, 0)),
        ],
        out_specs=pl.BlockSpec((1, N, BH, W), lambda b, hb: (b, 0, hb, 0)),
        out_shape=jax.ShapeDtypeStruct((B, N, H, W), jnp.float32),
    )(act8, ls, mask_logits, ff_highres_features)


# speedup vs baseline: 2.0714x; 2.0714x over previous
import jax
import jax.numpy as jnp
from jax.experimental import pallas as pl
from jax.experimental.pallas import tpu as pltpu

TS = 16
B, N, H, W = 2, 8, 384, 384
C = 96
K = 128
TW = W // TS
GW = 3
BH = 32
NBLK = H // BH
TR = BH // TS


def _dense_body(act8_ref, ls_ref, mask_ref, ff_ref, out_ref):
    b = pl.program_id(0)
    hb = pl.program_id(1)
    x = ls_ref[0]
    strength = jnp.maximum(x, 0.0) + jnp.log(1.0 + jnp.exp(-jnp.abs(x)))
    sig = jnp.tanh(jnp.sum(ff_ref[0], axis=0) * (1.0 / C))
    j = jax.lax.broadcasted_iota(jnp.int32, (TS, 128), 1) // TS
    for tr in range(TR):
        for g in range(GW):
            bits = act8_ref[b, (hb * TR + tr) * GW + g]
            rs = slice(tr * TS, (tr + 1) * TS)
            cs = slice(g * 128, (g + 1) * 128)
            active = ((bits >> j) & 1) == 1
            delta = jnp.where(active, strength * sig[rs, cs], 0.0)
            out_ref[0, :, rs, cs] = mask_ref[0, :, rs, cs] + delta[None, :, :]


def kernel(mask_logits, ff_highres_features, log_strength, active_tile_indices):
    idx = jnp.asarray(active_tile_indices, jnp.int32)
    ls = jnp.asarray(log_strength, jnp.float32).reshape(1)
    tids = jnp.arange((H // TS) * TW, dtype=jnp.int32)
    tile_act = jnp.any(idx[:, None, :] == tids[None, :, None], axis=-1).astype(
        jnp.int32
    )
    act8 = jnp.sum(
        tile_act.reshape(B, H // TS, GW, 8) << jnp.arange(8, dtype=jnp.int32),
        axis=-1,
    ).reshape(B, (H // TS) * GW)

    return pl.pallas_call(
        _dense_body,
        grid=(B, NBLK),
        in_specs=[
            pl.BlockSpec(memory_space=pltpu.SMEM),
            pl.BlockSpec(memory_space=pltpu.SMEM),
            pl.BlockSpec((1, N, BH, W), lambda b, hb: (b, 0, hb, 0)),
            pl.BlockSpec((1, C, BH, W), lambda b, hb: (b, 0, hb, 0)),
        ],
        out_specs=pl.BlockSpec((1, N, BH, W), lambda b, hb: (b, 0, hb, 0)),
        out_shape=jax.ShapeDtypeStruct((B, N, H, W), jnp.float32),
    )(act8, ls, mask_logits, ff_highres_features)
